# baseline (speedup 1.0000x reference)
import functools

import jax
import jax.numpy as jnp
import numpy as np
from jax.experimental import pallas as pl
from jax.experimental.pallas import tpu as pltpu
from jax.experimental.pallas import tpu_sc as plsc

BN = 512
BK = 1024


def _vq_tc(xt_ref, cb_ref, c2_ref, idx_ref, mind_ref, rs_ref, rq_ref,
           min_s, arg_s):
    j = pl.program_id(1)
    nk = pl.num_programs(1)
    xt = xt_ref[...]
    cb = cb_ref[...]
    c2 = c2_ref[...]
    x2 = jnp.sum(xt * xt, axis=1)
    mm = jax.lax.dot_general(xt, cb, (((1,), (1,)), ((), ())),
                             preferred_element_type=jnp.float32)
    s = (x2[:, None] - 2.0 * mm) + c2[None, :]
    lmin = jnp.min(s, axis=1)
    col = jax.lax.broadcasted_iota(jnp.int32, s.shape, 1)
    larg = jnp.min(jnp.where(s == lmin[:, None], col, BK), axis=1) + j * BK

    @pl.when(j == 0)
    def _():
        min_s[...] = lmin
        arg_s[...] = larg
        rs_ref[...] = jnp.sum(xt, axis=1)
        rq_ref[...] = x2

    @pl.when(j > 0)
    def _():
        pm = min_s[...]
        upd = lmin < pm
        arg_s[...] = jnp.where(upd, larg, arg_s[...])
        min_s[...] = jnp.where(upd, lmin, pm)

    @pl.when(j == nk - 1)
    def _():
        idx_ref[...] = arg_s[...]
        mind_ref[...] = min_s[...]


def _tc_call(xt, codebook, c2):
    n, e = xt.shape
    k = codebook.shape[0]
    return pl.pallas_call(
        _vq_tc,
        grid=(n // BN, k // BK),
        in_specs=[
            pl.BlockSpec((BN, e), lambda i, j: (i, 0)),
            pl.BlockSpec((BK, e), lambda i, j: (j, 0)),
            pl.BlockSpec((BK,), lambda i, j: (j,)),
        ],
        out_specs=[
            pl.BlockSpec((BN,), lambda i, j: (i,)),
            pl.BlockSpec((BN,), lambda i, j: (i,)),
            pl.BlockSpec((BN,), lambda i, j: (i,)),
            pl.BlockSpec((BN,), lambda i, j: (i,)),
        ],
        out_shape=[
            jax.ShapeDtypeStruct((n,), jnp.int32),
            jax.ShapeDtypeStruct((n,), jnp.float32),
            jax.ShapeDtypeStruct((n,), jnp.float32),
            jax.ShapeDtypeStruct((n,), jnp.float32),
        ],
        scratch_shapes=[
            pltpu.VMEM((BN,), jnp.float32),
            pltpu.VMEM((BN,), jnp.int32),
        ],
    )(xt, codebook, c2)


def _sc_gather(codebook, idxs):
    v, d = codebook.shape
    b = idxs.shape[0]
    info = plsc.get_sparse_core_info()
    nw = info.num_cores * info.num_subcores
    bpw = b // nw
    mesh = plsc.VectorSubcoreMesh(core_axis_name="c", subcore_axis_name="s")

    @functools.partial(
        pl.kernel, mesh=mesh,
        out_type=jax.ShapeDtypeStruct((b, d), jnp.float32),
        scratch_types=[
            pltpu.VMEM((bpw,), jnp.int32),
            pltpu.VMEM((bpw, d), jnp.float32),
            pltpu.SemaphoreType.DMA,
        ],
    )
    def k(table_hbm, idx_hbm, out_hbm, idx_v, rows_v, sem):
        wid = jax.lax.axis_index("s") * info.num_cores + jax.lax.axis_index("c")
        base = wid * bpw
        pltpu.sync_copy(idx_hbm.at[pl.ds(base, bpw)], idx_v)
        pltpu.async_copy(table_hbm.at[idx_v], rows_v, sem).wait()
        pltpu.sync_copy(rows_v, out_hbm.at[pl.ds(base, bpw)])

    return k(codebook, idxs)


def kernel(x, codebook):
    b, e, t = x.shape
    n = b * t
    xt = jnp.transpose(x, (0, 2, 1)).reshape(n, e)
    c2 = jnp.sum(codebook.T ** 2, axis=0)
    idxs, mind, rs, rq = _tc_call(xt, codebook, c2)
    out_flat = _sc_gather(codebook, idxs)
    denom = float(n * e)
    fit = jnp.mean(mind)
    commit_loss = jnp.sum(mind) / denom
    sx = jnp.sum(rs)
    sq = jnp.sum(rq)
    x_norm = jnp.sqrt(jnp.maximum(sq - sx * sx / denom, 0.0)) / np.sqrt(denom)
    out = jnp.transpose(out_flat.reshape(b, t, e), (0, 2, 1))
    return out, commit_loss, fit, x_norm


# speedup vs baseline: 1.1662x; 1.1662x over previous
import functools

import jax
import jax.numpy as jnp
import numpy as np
from jax.experimental import pallas as pl
from jax.experimental.pallas import tpu as pltpu
from jax.experimental.pallas import tpu_sc as plsc

BN = 256
BK = 2048
LANES = 128


def _vq_tc(xt_ref, cb2t_ref, c2_ref, idx_ref, mind_ref, rs_ref, rq_ref,
           amin_s, aidx_s, x2b_s):
    j = pl.program_id(1)
    nk = pl.num_programs(1)
    xt = xt_ref[...]
    cb2t = cb2t_ref[...]
    c2 = c2_ref[...]

    @pl.when(j == 0)
    def _():
        x2 = jnp.sum(xt * xt, axis=1)
        x2b_s[...] = jnp.broadcast_to(x2[:, None], (BN, LANES))
        amin_s[...] = jnp.full((BN, LANES), jnp.inf, jnp.float32)
        rs_ref[...] = jnp.sum(xt, axis=1)
        rq_ref[...] = x2

    mm2 = jnp.dot(xt, cb2t, preferred_element_type=jnp.float32)
    x2b = x2b_s[...]
    amin = amin_s[...]
    aidx = aidx_s[...]
    base0 = j * BK
    for c in range(BK // LANES):
        mmc = mm2[:, c * LANES:(c + 1) * LANES]
        c2c = jnp.broadcast_to(c2[:, c * LANES:(c + 1) * LANES], (BN, LANES))
        sc = (x2b - mmc) + c2c
        upd = sc < amin
        amin = jnp.minimum(amin, sc)
        aidx = jnp.where(upd, base0 + c * LANES, aidx)
    amin_s[...] = amin
    aidx_s[...] = aidx

    @pl.when(j == nk - 1)
    def _():
        gmin = jnp.min(amin, axis=1)
        lane = jax.lax.broadcasted_iota(jnp.int32, (BN, LANES), 1)
        fidx = aidx + lane
        cand = jnp.where(amin == gmin[:, None], fidx, jnp.int32(BK * nk))
        idx_ref[...] = jnp.min(cand, axis=1)
        mind_ref[...] = gmin


def _tc_call(xt, cb2t, c2):
    n, e = xt.shape
    k = cb2t.shape[1]
    return pl.pallas_call(
        _vq_tc,
        grid=(n // BN, k // BK),
        in_specs=[
            pl.BlockSpec((BN, e), lambda i, j: (i, 0)),
            pl.BlockSpec((e, BK), lambda i, j: (0, j)),
            pl.BlockSpec((1, BK), lambda i, j: (0, j)),
        ],
        out_specs=[
            pl.BlockSpec((BN,), lambda i, j: (i,)),
            pl.BlockSpec((BN,), lambda i, j: (i,)),
            pl.BlockSpec((BN,), lambda i, j: (i,)),
            pl.BlockSpec((BN,), lambda i, j: (i,)),
        ],
        out_shape=[
            jax.ShapeDtypeStruct((n,), jnp.int32),
            jax.ShapeDtypeStruct((n,), jnp.float32),
            jax.ShapeDtypeStruct((n,), jnp.float32),
            jax.ShapeDtypeStruct((n,), jnp.float32),
        ],
        scratch_shapes=[
            pltpu.VMEM((BN, LANES), jnp.float32),
            pltpu.VMEM((BN, LANES), jnp.int32),
            pltpu.VMEM((BN, LANES), jnp.float32),
        ],
    )(xt, cb2t, c2)


def _sc_gather(codebook, idxs):
    v, d = codebook.shape
    b = idxs.shape[0]
    info = plsc.get_sparse_core_info()
    nw = info.num_cores * info.num_subcores
    bpw = b // nw
    mesh = plsc.VectorSubcoreMesh(core_axis_name="c", subcore_axis_name="s")

    @functools.partial(
        pl.kernel, mesh=mesh,
        out_type=jax.ShapeDtypeStruct((b, d), jnp.float32),
        scratch_types=[
            pltpu.VMEM((bpw,), jnp.int32),
            pltpu.VMEM((bpw, d), jnp.float32),
            pltpu.SemaphoreType.DMA,
        ],
    )
    def k(table_hbm, idx_hbm, out_hbm, idx_v, rows_v, sem):
        wid = jax.lax.axis_index("s") * info.num_cores + jax.lax.axis_index("c")
        base = wid * bpw
        pltpu.sync_copy(idx_hbm.at[pl.ds(base, bpw)], idx_v)
        pltpu.async_copy(table_hbm.at[idx_v], rows_v, sem).wait()
        pltpu.sync_copy(rows_v, out_hbm.at[pl.ds(base, bpw)])

    return k(codebook, idxs)


def kernel(x, codebook):
    b, e, t = x.shape
    n = b * t
    xt = jnp.transpose(x, (0, 2, 1)).reshape(n, e)
    cb2t = jnp.transpose(codebook + codebook)
    c2 = jnp.sum(codebook.T ** 2, axis=0)[None, :]
    idxs, mind, rs, rq = _tc_call(xt, cb2t, c2)
    out_flat = _sc_gather(codebook, idxs)
    denom = float(n * e)
    fit = jnp.mean(mind)
    commit_loss = jnp.sum(mind) / denom
    sx = jnp.sum(rs)
    sq = jnp.sum(rq)
    x_norm = jnp.sqrt(jnp.maximum(sq - sx * sx / denom, 0.0)) / np.sqrt(denom)
    out = jnp.transpose(out_flat.reshape(b, t, e), (0, 2, 1))
    return out, commit_loss, fit, x_norm


# speedup vs baseline: 1.7100x; 1.4662x over previous
import functools

import jax
import jax.numpy as jnp
import numpy as np
from jax.experimental import pallas as pl
from jax.experimental.pallas import tpu as pltpu
from jax.experimental.pallas import tpu_sc as plsc

BN = 256
BK = 2048
LANES = 128


def _vq_tc(xt_ref, cb2t_ref, c2_ref, idx_ref, mind_ref, rs_ref, rq_ref):
    k = cb2t_ref.shape[1]
    xt = xt_ref[...]
    c2 = c2_ref[...]
    x2 = jnp.sum(xt * xt, axis=1)
    x2b = jnp.broadcast_to(x2[:, None], (BN, LANES))
    rs_ref[...] = jnp.sum(xt, axis=1)
    rq_ref[...] = x2
    amin = jnp.full((BN, LANES), jnp.inf, jnp.float32)
    aidx = jnp.zeros((BN, LANES), jnp.int32)
    for j in range(k // BK):
        cb2t = cb2t_ref[:, j * BK:(j + 1) * BK]
        mm2 = jnp.dot(xt, cb2t, preferred_element_type=jnp.float32)
        for c in range(BK // LANES):
            mmc = mm2[:, c * LANES:(c + 1) * LANES]
            c2c = jnp.broadcast_to(
                c2[:, j * BK + c * LANES:j * BK + (c + 1) * LANES], (BN, LANES))
            sc = (x2b - mmc) + c2c
            upd = sc < amin
            amin = jnp.minimum(amin, sc)
            aidx = jnp.where(upd, j * BK + c * LANES, aidx)
    gmin = jnp.min(amin, axis=1)
    lane = jax.lax.broadcasted_iota(jnp.int32, (BN, LANES), 1)
    fidx = aidx + lane
    cand = jnp.where(amin == gmin[:, None], fidx, jnp.int32(k))
    idx_ref[...] = jnp.min(cand, axis=1)
    mind_ref[...] = gmin


def _tc_call(xt, cb2t, c2):
    n, e = xt.shape
    k = cb2t.shape[1]
    return pl.pallas_call(
        _vq_tc,
        grid=(n // BN,),
        in_specs=[
            pl.BlockSpec((BN, e), lambda i: (i, 0)),
            pl.BlockSpec((e, k), lambda i: (0, 0)),
            pl.BlockSpec((1, k), lambda i: (0, 0)),
        ],
        out_specs=[
            pl.BlockSpec((BN,), lambda i: (i,)),
            pl.BlockSpec((BN,), lambda i: (i,)),
            pl.BlockSpec((BN,), lambda i: (i,)),
            pl.BlockSpec((BN,), lambda i: (i,)),
        ],
        out_shape=[
            jax.ShapeDtypeStruct((n,), jnp.int32),
            jax.ShapeDtypeStruct((n,), jnp.float32),
            jax.ShapeDtypeStruct((n,), jnp.float32),
            jax.ShapeDtypeStruct((n,), jnp.float32),
        ],
    )(xt, cb2t, c2)


def _sc_gather(codebook, idxs):
    v, d = codebook.shape
    b = idxs.shape[0]
    info = plsc.get_sparse_core_info()
    nw = info.num_cores * info.num_subcores
    bpw = b // nw
    mesh = plsc.VectorSubcoreMesh(core_axis_name="c", subcore_axis_name="s")

    @functools.partial(
        pl.kernel, mesh=mesh,
        out_type=jax.ShapeDtypeStruct((b, d), jnp.float32),
        scratch_types=[
            pltpu.VMEM((bpw,), jnp.int32),
            pltpu.VMEM((bpw, d), jnp.float32),
            pltpu.SemaphoreType.DMA,
        ],
    )
    def k(table_hbm, idx_hbm, out_hbm, idx_v, rows_v, sem):
        wid = jax.lax.axis_index("s") * info.num_cores + jax.lax.axis_index("c")
        base = wid * bpw
        pltpu.sync_copy(idx_hbm.at[pl.ds(base, bpw)], idx_v)
        pltpu.async_copy(table_hbm.at[idx_v], rows_v, sem).wait()
        pltpu.sync_copy(rows_v, out_hbm.at[pl.ds(base, bpw)])

    return k(codebook, idxs)


def kernel(x, codebook):
    b, e, t = x.shape
    n = b * t
    xt = jnp.transpose(x, (0, 2, 1)).reshape(n, e)
    cb2t = jnp.transpose(codebook + codebook)
    c2 = jnp.sum(codebook.T ** 2, axis=0)[None, :]
    idxs, mind, rs, rq = _tc_call(xt, cb2t, c2)
    out_flat = _sc_gather(codebook, idxs)
    denom = float(n * e)
    fit = jnp.mean(mind)
    commit_loss = jnp.sum(mind) / denom
    sx = jnp.sum(rs)
    sq = jnp.sum(rq)
    x_norm = jnp.sqrt(jnp.maximum(sq - sx * sx / denom, 0.0)) / np.sqrt(denom)
    out = jnp.transpose(out_flat.reshape(b, t, e), (0, 2, 1))
    return out, commit_loss, fit, x_norm
